# baseline (speedup 1.0000x reference)
import functools

import jax
import jax.numpy as jnp
from jax.experimental import pallas as pl
from jax.experimental.pallas import tpu as pltpu

N = 10000
D = 128
H = 128
G = 64
EPS = 1e-5

TM = 400
NT = N // TM
BF = jnp.bfloat16


def _fused_kernel(x_ref, adj_ref, w1_ref, b1_ref, w2_ref, b2_ref,
                  g1_ref, be1_ref, g_ref, be_ref, gpt_ref,
                  h_ref, ph_ref,
                  z_acc, s1, ss1, s2, ss2):
    p = pl.program_id(0)
    i = pl.program_id(1)
    rows = pl.ds(i * TM, TM)

    @pl.when(p == 0)
    def _phase0():
        pooled = jnp.dot(adj_ref[...].astype(BF), x_ref[...],
                         preferred_element_type=jnp.float32)
        z = jnp.dot(pooled, w1_ref[...], preferred_element_type=jnp.float32)
        z = z + b1_ref[...]
        z_acc[rows, :] = z

        @pl.when(i == 0)
        def _init():
            s1[...] = jnp.zeros_like(s1)
            ss1[...] = jnp.zeros_like(ss1)

        s1[...] += jnp.sum(z, axis=0, keepdims=True)
        ss1[...] += jnp.sum(z * z, axis=0, keepdims=True)

    @pl.when(p == 1)
    def _phase1():
        m = s1[...] / N
        v = ss1[...] / N - m * m
        scale = g1_ref[...] * jax.lax.rsqrt(v + EPS)
        a = jax.nn.relu((z_acc[rows, :] - m) * scale + be1_ref[...])
        r = jnp.dot(a.astype(BF), w2_ref[...].astype(BF),
                    preferred_element_type=jnp.float32) + b2_ref[...]
        z_acc[rows, :] = r

        @pl.when(i == 0)
        def _init():
            s2[...] = jnp.zeros_like(s2)
            ss2[...] = jnp.zeros_like(ss2)

        s2[...] += jnp.sum(r, axis=0, keepdims=True)
        ss2[...] += jnp.sum(r * r, axis=0, keepdims=True)

    @pl.when(p == 2)
    def _phase2():
        m = s2[...] / N
        v = ss2[...] / N - m * m
        scale = g_ref[...] * jax.lax.rsqrt(v + EPS)
        h = jax.nn.relu((z_acc[rows, :] - m) * scale + be_ref[...])
        h_ref[...] = h

        @pl.when(i == 0)
        def _init():
            ph_ref[...] = jnp.zeros_like(ph_ref)

        ph_ref[...] += jax.lax.dot_general(
            gpt_ref[...].astype(BF), h.astype(BF), (((0,), (0,)), ((), ())),
            preferred_element_type=jnp.float32)


@functools.partial(jax.jit, static_argnames=("interpret",))
def kernel(x, graph_pool, padded_nei, adj, W1_0, b1_0, W2_0, b2_0,
           g1_0, be1_0, g_0, be_0, interpret=False):
    del padded_nei
    b1 = b1_0.reshape(1, H)
    b2 = b2_0.reshape(1, H)
    g1 = g1_0.reshape(1, H)
    be1 = be1_0.reshape(1, H)
    g = g_0.reshape(1, H)
    be = be_0.reshape(1, H)
    x16 = x.astype(BF)
    gpt = graph_pool.T

    last = NT - 1

    h_nodes, pooled_h = pl.pallas_call(
        _fused_kernel,
        grid=(3, NT),
        in_specs=[
            pl.BlockSpec((N, D), lambda p, i: (0, 0)),
            pl.BlockSpec((TM, N), lambda p, i: (jnp.where(p == 0, i, last), 0)),
            pl.BlockSpec((D, H), lambda p, i: (0, 0)),
            pl.BlockSpec((1, H), lambda p, i: (0, 0)),
            pl.BlockSpec((H, H), lambda p, i: (0, 0)),
            pl.BlockSpec((1, H), lambda p, i: (0, 0)),
            pl.BlockSpec((1, H), lambda p, i: (0, 0)),
            pl.BlockSpec((1, H), lambda p, i: (0, 0)),
            pl.BlockSpec((1, H), lambda p, i: (0, 0)),
            pl.BlockSpec((1, H), lambda p, i: (0, 0)),
            pl.BlockSpec((TM, G), lambda p, i: (jnp.where(p == 2, i, 0), 0)),
        ],
        out_specs=[
            pl.BlockSpec((TM, H), lambda p, i: (jnp.where(p == 2, i, 0), 0)),
            pl.BlockSpec((G, H), lambda p, i: (0, 0)),
        ],
        out_shape=[
            jax.ShapeDtypeStruct((N, H), jnp.float32),
            jax.ShapeDtypeStruct((G, H), jnp.float32),
        ],
        scratch_shapes=[
            pltpu.VMEM((N, H), jnp.float32),
            pltpu.VMEM((1, H), jnp.float32),
            pltpu.VMEM((1, H), jnp.float32),
            pltpu.VMEM((1, H), jnp.float32),
            pltpu.VMEM((1, H), jnp.float32),
        ],
        interpret=interpret,
    )(x16, adj, W1_0, b1, W2_0, b2, g1, be1, g, be, gpt)

    return (pooled_h, h_nodes)


# speedup vs baseline: 1.1021x; 1.1021x over previous
import functools

import jax
import jax.numpy as jnp
from jax.experimental import pallas as pl
from jax.experimental.pallas import tpu as pltpu

N = 10000
D = 128
H = 128
G = 64
EPS = 1e-5

TM = 400
NT = N // TM
TM2 = 2000
NT2 = N // TM2
BF = jnp.bfloat16


def _fused_kernel(x_ref, adj_ref, w1_ref, b1_ref, w2_ref, b2_ref,
                  g1_ref, be1_ref, g_ref, be_ref, gpt_ref,
                  h_ref, ph_ref,
                  acc, sc2, sh2):
    g = pl.program_id(0)

    @pl.when(g < NT)
    def _stream():
        rows = pl.ds(jnp.minimum(g, NT - 1) * TM, TM)
        acc[rows, :] = jnp.dot(adj_ref[...].astype(BF), x_ref[...],
                               preferred_element_type=jnp.float32)

    @pl.when(g == NT)
    def _mlp():
        z = jnp.dot(acc[...].astype(BF), w1_ref[...].astype(BF),
                    preferred_element_type=jnp.float32) + b1_ref[...]
        m = jnp.mean(z, axis=0, keepdims=True)
        v = jnp.mean(z * z, axis=0, keepdims=True) - m * m
        scale = g1_ref[...] * jax.lax.rsqrt(v + EPS)
        a = jax.nn.relu((z - m) * scale + be1_ref[...])
        r = jnp.dot(a.astype(BF), w2_ref[...].astype(BF),
                    preferred_element_type=jnp.float32) + b2_ref[...]
        m2 = jnp.mean(r, axis=0, keepdims=True)
        v2 = jnp.mean(r * r, axis=0, keepdims=True) - m2 * m2
        s2 = g_ref[...] * jax.lax.rsqrt(v2 + EPS)
        sc2[...] = s2
        sh2[...] = be_ref[...] - m2 * s2
        acc[...] = r

    @pl.when(g > NT)
    def _epilogue():
        j = jnp.minimum(g - (NT + 1), NT2 - 1)
        rows = pl.ds(j * TM2, TM2)
        h = jax.nn.relu(acc[rows, :] * sc2[...] + sh2[...])
        h_ref[...] = h

        @pl.when(g == NT + 1)
        def _init():
            ph_ref[...] = jnp.zeros_like(ph_ref)

        ph_ref[...] += jax.lax.dot_general(
            gpt_ref[...].astype(BF), h.astype(BF), (((0,), (0,)), ((), ())),
            preferred_element_type=jnp.float32)


@functools.partial(jax.jit, static_argnames=("interpret",))
def kernel(x, graph_pool, padded_nei, adj, W1_0, b1_0, W2_0, b2_0,
           g1_0, be1_0, g_0, be_0, interpret=False):
    del padded_nei
    b1 = b1_0.reshape(1, H)
    b2 = b2_0.reshape(1, H)
    g1 = g1_0.reshape(1, H)
    be1 = be1_0.reshape(1, H)
    g = g_0.reshape(1, H)
    be = be_0.reshape(1, H)
    x16 = x.astype(BF)
    gpt = graph_pool.T

    adj_last = NT - 1

    def adj_map(gg, last=adj_last):
        return (jnp.minimum(gg, last), 0)

    def epi_map(gg):
        return (jnp.clip(gg - (NT + 1), 0, NT2 - 1), 0)

    h_nodes, pooled_h = pl.pallas_call(
        _fused_kernel,
        grid=(NT + 1 + NT2,),
        in_specs=[
            pl.BlockSpec((N, D), lambda gg: (0, 0)),
            pl.BlockSpec((TM, N), adj_map),
            pl.BlockSpec((D, H), lambda gg: (0, 0)),
            pl.BlockSpec((1, H), lambda gg: (0, 0)),
            pl.BlockSpec((H, H), lambda gg: (0, 0)),
            pl.BlockSpec((1, H), lambda gg: (0, 0)),
            pl.BlockSpec((1, H), lambda gg: (0, 0)),
            pl.BlockSpec((1, H), lambda gg: (0, 0)),
            pl.BlockSpec((1, H), lambda gg: (0, 0)),
            pl.BlockSpec((1, H), lambda gg: (0, 0)),
            pl.BlockSpec((TM2, G), epi_map),
        ],
        out_specs=[
            pl.BlockSpec((TM2, H), epi_map),
            pl.BlockSpec((G, H), lambda gg: (0, 0)),
        ],
        out_shape=[
            jax.ShapeDtypeStruct((N, H), jnp.float32),
            jax.ShapeDtypeStruct((G, H), jnp.float32),
        ],
        scratch_shapes=[
            pltpu.VMEM((N, H), jnp.float32),
            pltpu.VMEM((1, H), jnp.float32),
            pltpu.VMEM((1, H), jnp.float32),
        ],
        interpret=interpret,
    )(x16, adj, W1_0, b1, W2_0, b2, g1, be1, g, be, gpt)

    return (pooled_h, h_nodes)


# speedup vs baseline: 1.1237x; 1.0196x over previous
import functools

import jax
import jax.numpy as jnp
from jax.experimental import pallas as pl
from jax.experimental.pallas import tpu as pltpu

N = 10000
D = 128
H = 128
G = 64
EPS = 1e-5

TM = 400
NT = N // TM
TM2 = 5000
NT2 = N // TM2
BF = jnp.bfloat16


def _fused_kernel(x_ref, adj_ref, w1_ref, b1_ref, w2_ref, b2_ref,
                  g1_ref, be1_ref, g_ref, be_ref, gpt_ref,
                  h_ref, ph_ref,
                  acc, xbf, sc2, sh2):
    g = pl.program_id(0)

    @pl.when(g == 0)
    def _cast_x():
        xbf[...] = x_ref[...].astype(BF)

    @pl.when(g < NT)
    def _stream():
        rows = pl.ds(jnp.minimum(g, NT - 1) * TM, TM)
        acc[rows, :] = jnp.dot(adj_ref[...].astype(BF), xbf[...],
                               preferred_element_type=jnp.float32)

    @pl.when(g == NT)
    def _mlp():
        z = jnp.dot(acc[...].astype(BF), w1_ref[...].astype(BF),
                    preferred_element_type=jnp.float32) + b1_ref[...]
        m = jnp.mean(z, axis=0, keepdims=True)
        v = jnp.mean(z * z, axis=0, keepdims=True) - m * m
        scale = g1_ref[...] * jax.lax.rsqrt(v + EPS)
        a = jax.nn.relu((z - m) * scale + be1_ref[...])
        r = jnp.dot(a.astype(BF), w2_ref[...].astype(BF),
                    preferred_element_type=jnp.float32) + b2_ref[...]
        m2 = jnp.mean(r, axis=0, keepdims=True)
        v2 = jnp.mean(r * r, axis=0, keepdims=True) - m2 * m2
        s2 = g_ref[...] * jax.lax.rsqrt(v2 + EPS)
        sc2[...] = s2
        sh2[...] = be_ref[...] - m2 * s2
        acc[...] = r

    @pl.when(g > NT)
    def _epilogue():
        j = jnp.minimum(g - (NT + 1), NT2 - 1)
        rows = pl.ds(j * TM2, TM2)
        h = jax.nn.relu(acc[rows, :] * sc2[...] + sh2[...])
        h_ref[...] = h

        @pl.when(g == NT + 1)
        def _init():
            ph_ref[...] = jnp.zeros_like(ph_ref)

        ph_ref[...] += jax.lax.dot_general(
            gpt_ref[...].astype(BF), h.astype(BF), (((0,), (0,)), ((), ())),
            preferred_element_type=jnp.float32)


@functools.partial(jax.jit, static_argnames=("interpret",))
def kernel(x, graph_pool, padded_nei, adj, W1_0, b1_0, W2_0, b2_0,
           g1_0, be1_0, g_0, be_0, interpret=False):
    del padded_nei
    b1 = b1_0.reshape(1, H)
    b2 = b2_0.reshape(1, H)
    g1 = g1_0.reshape(1, H)
    be1 = be1_0.reshape(1, H)
    g = g_0.reshape(1, H)
    be = be_0.reshape(1, H)
    gpt = graph_pool.T

    adj_last = NT - 1

    def adj_map(gg, last=adj_last):
        return (jnp.minimum(gg, last), 0)

    def epi_map(gg):
        return (jnp.clip(gg - (NT + 1), 0, NT2 - 1), 0)

    h_nodes, pooled_h = pl.pallas_call(
        _fused_kernel,
        grid=(NT + 1 + NT2,),
        in_specs=[
            pl.BlockSpec((N, D), lambda gg: (0, 0)),
            pl.BlockSpec((TM, N), adj_map),
            pl.BlockSpec((D, H), lambda gg: (0, 0)),
            pl.BlockSpec((1, H), lambda gg: (0, 0)),
            pl.BlockSpec((H, H), lambda gg: (0, 0)),
            pl.BlockSpec((1, H), lambda gg: (0, 0)),
            pl.BlockSpec((1, H), lambda gg: (0, 0)),
            pl.BlockSpec((1, H), lambda gg: (0, 0)),
            pl.BlockSpec((1, H), lambda gg: (0, 0)),
            pl.BlockSpec((1, H), lambda gg: (0, 0)),
            pl.BlockSpec((TM2, G), epi_map),
        ],
        out_specs=[
            pl.BlockSpec((TM2, H), epi_map),
            pl.BlockSpec((G, H), lambda gg: (0, 0)),
        ],
        out_shape=[
            jax.ShapeDtypeStruct((N, H), jnp.float32),
            jax.ShapeDtypeStruct((G, H), jnp.float32),
        ],
        scratch_shapes=[
            pltpu.VMEM((N, H), jnp.float32),
            pltpu.VMEM((N, D), BF),
            pltpu.VMEM((1, H), jnp.float32),
            pltpu.VMEM((1, H), jnp.float32),
        ],
        interpret=interpret,
    )(x, adj, W1_0, b1, W2_0, b2, g1, be1, g, be, gpt)

    return (pooled_h, h_nodes)


# speedup vs baseline: 1.1711x; 1.0422x over previous
import functools

import jax
import jax.numpy as jnp
from jax.experimental import pallas as pl
from jax.experimental.pallas import tpu as pltpu

N = 10000
D = 128
H = 128
G = 64
EPS = 1e-5

TM = 400
NT = N // TM
TM2 = 5000
NT2 = N // TM2
BF = jnp.bfloat16


def _fused_kernel(x_ref, adj_ref, w1_ref, b1_ref, w2_ref, b2_ref,
                  g1_ref, be1_ref, g_ref, be_ref, gp_ref,
                  h_ref, ph_ref,
                  acc, xbf, sc2, sh2):
    g = pl.program_id(0)

    @pl.when(g == 0)
    def _cast_x():
        xbf[...] = x_ref[...].astype(BF)

    @pl.when(g < NT)
    def _stream():
        rows = pl.ds(jnp.minimum(g, NT - 1) * TM, TM)
        acc[rows, :] = jnp.dot(adj_ref[...].astype(BF), xbf[...],
                               preferred_element_type=jnp.float32)

    @pl.when(g == NT)
    def _mlp():
        z = jnp.dot(acc[...].astype(BF), w1_ref[...].astype(BF),
                    preferred_element_type=jnp.float32) + b1_ref[...]
        m = jnp.mean(z, axis=0, keepdims=True)
        v = jnp.mean(z * z, axis=0, keepdims=True) - m * m
        scale = g1_ref[...] * jax.lax.rsqrt(v + EPS)
        a = jax.nn.relu((z - m) * scale + be1_ref[...])
        r = jnp.dot(a.astype(BF), w2_ref[...].astype(BF),
                    preferred_element_type=jnp.float32) + b2_ref[...]
        m2 = jnp.mean(r, axis=0, keepdims=True)
        v2 = jnp.mean(r * r, axis=0, keepdims=True) - m2 * m2
        s2 = g_ref[...] * jax.lax.rsqrt(v2 + EPS)
        sc2[...] = s2
        sh2[...] = be_ref[...] - m2 * s2
        acc[...] = r

    @pl.when((g > NT) & (g <= NT + NT2))
    def _epilogue():
        j = jnp.minimum(g - (NT + 1), NT2 - 1)
        rows = pl.ds(j * TM2, TM2)
        h = jax.nn.relu(acc[rows, :] * sc2[...] + sh2[...])
        h_ref[...] = h
        acc[rows, :] = h

    @pl.when(g == NT + NT2 + 1)
    def _readout():
        ph_ref[...] = jnp.dot(gp_ref[...].astype(BF), acc[...].astype(BF),
                              preferred_element_type=jnp.float32)


@functools.partial(jax.jit, static_argnames=("interpret",))
def kernel(x, graph_pool, padded_nei, adj, W1_0, b1_0, W2_0, b2_0,
           g1_0, be1_0, g_0, be_0, interpret=False):
    del padded_nei
    b1 = b1_0.reshape(1, H)
    b2 = b2_0.reshape(1, H)
    g1 = g1_0.reshape(1, H)
    be1 = be1_0.reshape(1, H)
    g = g_0.reshape(1, H)
    be = be_0.reshape(1, H)
    adj_last = NT - 1

    def adj_map(gg, last=adj_last):
        return (jnp.minimum(gg, last), 0)

    def epi_map(gg):
        return (jnp.clip(gg - (NT + 1), 0, NT2 - 1), 0)

    h_nodes, pooled_h = pl.pallas_call(
        _fused_kernel,
        grid=(NT + 1 + NT2 + 1,),
        in_specs=[
            pl.BlockSpec((N, D), lambda gg: (0, 0)),
            pl.BlockSpec((TM, N), adj_map),
            pl.BlockSpec((D, H), lambda gg: (0, 0)),
            pl.BlockSpec((1, H), lambda gg: (0, 0)),
            pl.BlockSpec((H, H), lambda gg: (0, 0)),
            pl.BlockSpec((1, H), lambda gg: (0, 0)),
            pl.BlockSpec((1, H), lambda gg: (0, 0)),
            pl.BlockSpec((1, H), lambda gg: (0, 0)),
            pl.BlockSpec((1, H), lambda gg: (0, 0)),
            pl.BlockSpec((1, H), lambda gg: (0, 0)),
            pl.BlockSpec((G, N), lambda gg: (0, 0)),
        ],
        out_specs=[
            pl.BlockSpec((TM2, H), epi_map),
            pl.BlockSpec((G, H), lambda gg: (0, 0)),
        ],
        out_shape=[
            jax.ShapeDtypeStruct((N, H), jnp.float32),
            jax.ShapeDtypeStruct((G, H), jnp.float32),
        ],
        scratch_shapes=[
            pltpu.VMEM((N, H), jnp.float32),
            pltpu.VMEM((N, D), BF),
            pltpu.VMEM((1, H), jnp.float32),
            pltpu.VMEM((1, H), jnp.float32),
        ],
        interpret=interpret,
    )(x, adj, W1_0, b1, W2_0, b2, g1, be1, g, be, graph_pool)

    return (pooled_h, h_nodes)


# speedup vs baseline: 1.1943x; 1.0198x over previous
import functools

import jax
import jax.numpy as jnp
from jax.experimental import pallas as pl
from jax.experimental.pallas import tpu as pltpu

N = 10000
D = 128
H = 128
G = 64
EPS = 1e-5

TM = 400
NT = N // TM
TM2 = 5000
NT2 = N // TM2
BF = jnp.bfloat16


def _fused_kernel(x_ref, adj_ref, w1_ref, w2_ref,
                  g1_ref, be1_ref, g_ref, be_ref, gp_ref,
                  h_ref, ph_ref,
                  acc, xbf, gpbf, s1, ss1, sc2, sh2):
    g = pl.program_id(0)

    @pl.when(g == 0)
    def _prologue():
        xbf[...] = x_ref[...].astype(BF)
        gpbf[...] = gp_ref[...].astype(BF)
        s1[...] = jnp.zeros_like(s1)
        ss1[...] = jnp.zeros_like(ss1)

    @pl.when(g < NT)
    def _stream():
        rows = pl.ds(jnp.minimum(g, NT - 1) * TM, TM)
        pooled = jnp.dot(adj_ref[...].astype(BF), xbf[...],
                         preferred_element_type=jnp.float32)
        z = jnp.dot(pooled.astype(BF), w1_ref[...].astype(BF),
                    preferred_element_type=jnp.float32)
        acc[rows, :] = z
        s1[...] += jnp.sum(z, axis=0, keepdims=True)
        ss1[...] += jnp.sum(z * z, axis=0, keepdims=True)

    @pl.when(g == NT)
    def _mlp():
        m = s1[...] / N
        v = ss1[...] / N - m * m
        sc1 = g1_ref[...] * jax.lax.rsqrt(v + EPS)
        sh1 = be1_ref[...] - m * sc1
        a = jax.nn.relu(acc[...] * sc1 + sh1)
        r = jnp.dot(a.astype(BF), w2_ref[...].astype(BF),
                    preferred_element_type=jnp.float32)
        m2 = jnp.mean(r, axis=0, keepdims=True)
        v2 = jnp.mean(r * r, axis=0, keepdims=True) - m2 * m2
        s2 = g_ref[...] * jax.lax.rsqrt(v2 + EPS)
        sc2[...] = s2
        sh2[...] = be_ref[...] - m2 * s2
        acc[...] = r

    @pl.when((g > NT) & (g <= NT + NT2))
    def _epilogue():
        j = jnp.minimum(g - (NT + 1), NT2 - 1)
        rows = pl.ds(j * TM2, TM2)
        h = jax.nn.relu(acc[rows, :] * sc2[...] + sh2[...])
        h_ref[...] = h
        acc[rows, :] = h

    @pl.when(g == NT + NT2 + 1)
    def _readout():
        ph_ref[...] = jnp.dot(gpbf[...], acc[...].astype(BF),
                              preferred_element_type=jnp.float32)


@functools.partial(jax.jit, static_argnames=("interpret",))
def kernel(x, graph_pool, padded_nei, adj, W1_0, b1_0, W2_0, b2_0,
           g1_0, be1_0, g_0, be_0, interpret=False):
    del padded_nei, b1_0, b2_0
    g1 = g1_0.reshape(1, H)
    be1 = be1_0.reshape(1, H)
    g = g_0.reshape(1, H)
    be = be_0.reshape(1, H)

    adj_last = NT - 1

    def adj_map(gg, last=adj_last):
        return (jnp.minimum(gg, last), 0)

    def epi_map(gg):
        return (jnp.clip(gg - (NT + 1), 0, NT2 - 1), 0)

    h_nodes, pooled_h = pl.pallas_call(
        _fused_kernel,
        grid=(NT + 1 + NT2 + 1,),
        in_specs=[
            pl.BlockSpec((N, D), lambda gg: (0, 0)),
            pl.BlockSpec((TM, N), adj_map),
            pl.BlockSpec((D, H), lambda gg: (0, 0)),
            pl.BlockSpec((H, H), lambda gg: (0, 0)),
            pl.BlockSpec((1, H), lambda gg: (0, 0)),
            pl.BlockSpec((1, H), lambda gg: (0, 0)),
            pl.BlockSpec((1, H), lambda gg: (0, 0)),
            pl.BlockSpec((1, H), lambda gg: (0, 0)),
            pl.BlockSpec((G, N), lambda gg: (0, 0)),
        ],
        out_specs=[
            pl.BlockSpec((TM2, H), epi_map),
            pl.BlockSpec((G, H), lambda gg: (0, 0)),
        ],
        out_shape=[
            jax.ShapeDtypeStruct((N, H), jnp.float32),
            jax.ShapeDtypeStruct((G, H), jnp.float32),
        ],
        scratch_shapes=[
            pltpu.VMEM((N, H), jnp.float32),
            pltpu.VMEM((N, D), BF),
            pltpu.VMEM((G, N), BF),
            pltpu.VMEM((1, H), jnp.float32),
            pltpu.VMEM((1, H), jnp.float32),
            pltpu.VMEM((1, H), jnp.float32),
            pltpu.VMEM((1, H), jnp.float32),
        ],
        interpret=interpret,
    )(x, adj, W1_0, W2_0, g1, be1, g, be, graph_pool)

    return (pooled_h, h_nodes)


# speedup vs baseline: 1.1965x; 1.0018x over previous
import functools

import jax
import jax.numpy as jnp
from jax.experimental import pallas as pl
from jax.experimental.pallas import tpu as pltpu

N = 10000
D = 128
H = 128
G = 64
EPS = 1e-5

TM = 400
NT = N // TM
TM2 = 5000
NT2 = N // TM2
BF = jnp.bfloat16


def _fused_kernel(x_ref, adj_ref, w1_ref, w2_ref,
                  g1_ref, be1_ref, g_ref, be_ref, gp_ref,
                  h_ref, ph_ref,
                  acc, xbf, gpbf, rbf, s1, ss1, sc2, sh2):
    g = pl.program_id(0)

    @pl.when(g == 0)
    def _prologue():
        xbf[...] = x_ref[...].astype(BF)
        gpbf[...] = gp_ref[...].astype(BF)
        s1[...] = jnp.zeros_like(s1)
        ss1[...] = jnp.zeros_like(ss1)

    @pl.when(g < NT)
    def _stream():
        rows = pl.ds(jnp.minimum(g, NT - 1) * TM, TM)
        pooled = jnp.dot(adj_ref[...].astype(BF), xbf[...],
                         preferred_element_type=jnp.float32)
        z = jnp.dot(pooled.astype(BF), w1_ref[...].astype(BF),
                    preferred_element_type=jnp.float32)
        acc[rows, :] = z
        s1[...] += jnp.sum(z, axis=0, keepdims=True)
        ss1[...] += jnp.sum(z * z, axis=0, keepdims=True)

    @pl.when(g == NT)
    def _mlp():
        m = s1[...] / N
        v = ss1[...] / N - m * m
        sc1 = g1_ref[...] * jax.lax.rsqrt(v + EPS)
        sh1 = be1_ref[...] - m * sc1
        a = jax.nn.relu(acc[...] * sc1 + sh1)
        a16 = a.astype(BF)
        w2 = w2_ref[...].astype(BF)
        r = jnp.dot(a16, w2, preferred_element_type=jnp.float32)
        rbf[...] = r.astype(BF)
        sa = jnp.sum(a, axis=0, keepdims=True)
        gram = jax.lax.dot_general(a16, a16, (((0,), (0,)), ((), ())),
                                   preferred_element_type=jnp.float32)
        m2 = jnp.dot(sa.astype(BF), w2, preferred_element_type=jnp.float32) / N
        v2 = jnp.sum(jnp.dot(gram, w2_ref[...],
                             preferred_element_type=jnp.float32) * w2_ref[...],
                     axis=0, keepdims=True) / N - m2 * m2
        s2 = g_ref[...] * jax.lax.rsqrt(v2 + EPS)
        sc2[...] = s2
        sh2[...] = be_ref[...] - m2 * s2

    @pl.when((g > NT) & (g <= NT + NT2))
    def _epilogue():
        j = jnp.minimum(g - (NT + 1), NT2 - 1)
        rows = pl.ds(j * TM2, TM2)
        h = jax.nn.relu(rbf[rows, :].astype(jnp.float32) * sc2[...] + sh2[...])
        h_ref[...] = h
        rbf[rows, :] = h.astype(BF)

    @pl.when(g == NT + NT2 + 1)
    def _readout():
        ph_ref[...] = jnp.dot(gpbf[...], rbf[...],
                              preferred_element_type=jnp.float32)


@functools.partial(jax.jit, static_argnames=("interpret",))
def kernel(x, graph_pool, padded_nei, adj, W1_0, b1_0, W2_0, b2_0,
           g1_0, be1_0, g_0, be_0, interpret=False):
    del padded_nei, b1_0, b2_0
    g1 = g1_0.reshape(1, H)
    be1 = be1_0.reshape(1, H)
    g = g_0.reshape(1, H)
    be = be_0.reshape(1, H)

    adj_last = NT - 1

    def adj_map(gg, last=adj_last):
        return (jnp.minimum(gg, last), 0)

    def epi_map(gg):
        return (jnp.clip(gg - (NT + 1), 0, NT2 - 1), 0)

    h_nodes, pooled_h = pl.pallas_call(
        _fused_kernel,
        grid=(NT + 1 + NT2 + 1,),
        in_specs=[
            pl.BlockSpec((N, D), lambda gg: (0, 0)),
            pl.BlockSpec((TM, N), adj_map),
            pl.BlockSpec((D, H), lambda gg: (0, 0)),
            pl.BlockSpec((H, H), lambda gg: (0, 0)),
            pl.BlockSpec((1, H), lambda gg: (0, 0)),
            pl.BlockSpec((1, H), lambda gg: (0, 0)),
            pl.BlockSpec((1, H), lambda gg: (0, 0)),
            pl.BlockSpec((1, H), lambda gg: (0, 0)),
            pl.BlockSpec((G, N), lambda gg: (0, 0)),
        ],
        out_specs=[
            pl.BlockSpec((TM2, H), epi_map),
            pl.BlockSpec((G, H), lambda gg: (0, 0)),
        ],
        out_shape=[
            jax.ShapeDtypeStruct((N, H), jnp.float32),
            jax.ShapeDtypeStruct((G, H), jnp.float32),
        ],
        scratch_shapes=[
            pltpu.VMEM((N, H), jnp.float32),
            pltpu.VMEM((N, D), BF),
            pltpu.VMEM((G, N), BF),
            pltpu.VMEM((N, H), BF),
            pltpu.VMEM((1, H), jnp.float32),
            pltpu.VMEM((1, H), jnp.float32),
            pltpu.VMEM((1, H), jnp.float32),
            pltpu.VMEM((1, H), jnp.float32),
        ],
        interpret=interpret,
    )(x, adj, W1_0, W2_0, g1, be1, g, be, graph_pool)

    return (pooled_h, h_nodes)
